# Initial kernel scaffold; baseline (speedup 1.0000x reference)
import jax
import jax.numpy as jnp
from jax.experimental import pallas as pl


def kernel(features, edge_index, W1_self, W1_neigh, b1, W2_self, W2_neigh, b2):
    raise NotImplementedError("write your pallas kernel here")


# speedup vs baseline: 3.0180x; 3.0180x over previous
import functools

import jax
import jax.numpy as jnp
from jax import lax
from jax.experimental import pallas as pl
from jax.experimental.pallas import tpu as pltpu
from jax.experimental.pallas import tpu_sc as plsc

N = 10000
E = 320000
D = 128

NC = 2
NS = 16
NW = NC * NS
K = 128
CH = 80
EPP = CH * K
EP = NW * EPP
NP = 10240
RPS = NP // NS
ZR = 128
BR = 1000
GRID = N // BR


def _sc_deg_body(dst_hbm, deg_out, deg_sh, dst_v, rows_v):
    c = lax.axis_index("c")
    s = lax.axis_index("s")
    wid = s * NC + c
    row0 = s * RPS

    zv = jnp.zeros((16,), jnp.float32)

    def fill_rows(val):
        def row(i, _):
            def inner(j, _):
                rows_v[i, pl.ds(j * 16, 16)] = val
                return 0
            return lax.fori_loop(0, D // 16, inner, 0)
        lax.fori_loop(0, K, row, 0)

    fill_rows(zv)
    for t in range(RPS // K):
        pltpu.sync_copy(rows_v, deg_sh.at[pl.ds(row0 + t * K, K)])
    plsc.subcore_barrier()

    fill_rows(jnp.ones((16,), jnp.float32))
    pltpu.sync_copy(dst_hbm.at[wid], dst_v)

    def step(i, _):
        pltpu.sync_copy(rows_v, deg_sh.at[dst_v.at[i]], add=True)
        return 0

    lax.fori_loop(0, CH, step, 0)
    plsc.subcore_barrier()

    pltpu.sync_copy(deg_sh.at[pl.ds(row0, RPS)],
                    deg_out.at[c, pl.ds(row0, RPS)])


def _sc_agg_body(y_hbm, src_hbm, dst_hbm, z_out,
                 z_sh, src_v, dst_v, rows_v, sem):
    c = lax.axis_index("c")
    s = lax.axis_index("s")
    wid = s * NC + c
    row0 = s * RPS

    zv = jnp.zeros((16,), jnp.float32)

    def zero_rows_row(i, _):
        def inner(j, _):
            rows_v[i, pl.ds(j * 16, 16)] = zv
            return 0
        return lax.fori_loop(0, D // 16, inner, 0)

    lax.fori_loop(0, K, zero_rows_row, 0)

    for t in range(RPS // K):
        pltpu.sync_copy(rows_v, z_sh.at[pl.ds(row0 + t * K, K)])
    plsc.subcore_barrier()

    pltpu.sync_copy(src_hbm.at[wid], src_v)
    pltpu.sync_copy(dst_hbm.at[wid], dst_v)

    def step(i, _):
        pltpu.async_copy(y_hbm.at[src_v.at[i]], rows_v, sem).wait()
        pltpu.sync_copy(rows_v, z_sh.at[dst_v.at[i]], add=True)
        return 0

    lax.fori_loop(0, CH, step, 0)
    plsc.subcore_barrier()

    pltpu.sync_copy(z_sh.at[pl.ds(row0, RPS)],
                    z_out.at[c, pl.ds(row0, RPS)])


def _sc_mesh():
    return plsc.VectorSubcoreMesh(core_axis_name="c", subcore_axis_name="s")


@functools.lru_cache(maxsize=None)
def _make_sc_deg():
    return pl.kernel(
        _sc_deg_body,
        mesh=_sc_mesh(),
        out_type=[jax.ShapeDtypeStruct((NC, NP, D), jnp.float32)],
        scratch_types=[
            pltpu.VMEM_SHARED((NP, D), jnp.float32),
            pltpu.VMEM((CH, K), jnp.int32),
            pltpu.VMEM((K, D), jnp.float32),
        ],
    )


@functools.lru_cache(maxsize=None)
def _make_sc_agg():
    return pl.kernel(
        _sc_agg_body,
        mesh=_sc_mesh(),
        out_type=[jax.ShapeDtypeStruct((NC, NP, D), jnp.float32)],
        scratch_types=[
            pltpu.VMEM_SHARED((NP, D), jnp.float32),
            pltpu.VMEM((CH, K), jnp.int32),
            pltpu.VMEM((CH, K), jnp.int32),
            pltpu.VMEM((K, D), jnp.float32),
            pltpu.SemaphoreType.DMA,
        ],
    )


def _tc_pre_body(x_ref, ws_ref, wn_ref, b_ref, s_out, y_out):
    x = x_ref[...]
    s_out[...] = (jnp.dot(x, ws_ref[...], preferred_element_type=jnp.float32)
                  + b_ref[0:1, :])
    y_out[...] = jnp.dot(x, wn_ref[...], preferred_element_type=jnp.float32)


def _tc_mid_body(s1_ref, z_ref, deg_ref, ws_ref, wn_ref, b_ref, s2_out, y2_out):
    deg = deg_ref[0, :, 0:1] + deg_ref[1, :, 0:1]
    inv = 1.0 / jnp.maximum(deg, 1.0)
    h = jnp.maximum(s1_ref[...] + (z_ref[0] + z_ref[1]) * inv, 0.0)
    s2_out[...] = (jnp.dot(h, ws_ref[...], preferred_element_type=jnp.float32)
                   + b_ref[0:1, :])
    y2_out[...] = jnp.dot(h, wn_ref[...], preferred_element_type=jnp.float32)


def _tc_post_body(s2_ref, z_ref, deg_ref, out_ref):
    deg = deg_ref[0, :, 0:1] + deg_ref[1, :, 0:1]
    inv = 1.0 / jnp.maximum(deg, 1.0)
    out_ref[...] = s2_ref[...] + (z_ref[0] + z_ref[1]) * inv


_row_spec = pl.BlockSpec((BR, D), lambda i: (i, 0))
_w_spec = pl.BlockSpec((D, D), lambda i: (0, 0))
_b_spec = pl.BlockSpec((8, D), lambda i: (0, 0))
_z_spec = pl.BlockSpec((NC, BR, D), lambda i: (0, i, 0))

_tc_pre = pl.pallas_call(
    _tc_pre_body,
    grid=(GRID,),
    in_specs=[_row_spec, _w_spec, _w_spec, _b_spec],
    out_specs=[_row_spec, _row_spec],
    out_shape=[jax.ShapeDtypeStruct((N, D), jnp.float32)] * 2,
)

_tc_mid = pl.pallas_call(
    _tc_mid_body,
    grid=(GRID,),
    in_specs=[_row_spec, _z_spec, _z_spec, _w_spec, _w_spec, _b_spec],
    out_specs=[_row_spec, _row_spec],
    out_shape=[jax.ShapeDtypeStruct((N, D), jnp.float32)] * 2,
)

_tc_post = pl.pallas_call(
    _tc_post_body,
    grid=(GRID,),
    in_specs=[_row_spec, _z_spec, _z_spec],
    out_specs=_row_spec,
    out_shape=jax.ShapeDtypeStruct((N, D), jnp.float32),
)


def kernel(features, edge_index, W1_self, W1_neigh, b1, W2_self, W2_neigh, b2):
    pad = EP - E
    src_r = jnp.concatenate(
        [edge_index[0], jnp.zeros((pad,), jnp.int32)]).reshape(NW, CH, K)
    dst_r = jnp.concatenate(
        [edge_index[1], jnp.full((pad,), N, jnp.int32)]).reshape(NW, CH, K)
    b1r = jnp.broadcast_to(b1.reshape(1, D), (8, D))
    b2r = jnp.broadcast_to(b2.reshape(1, D), (8, D))

    (deg,) = _make_sc_deg()(dst_r)
    s1, y1 = _tc_pre(features, W1_self, W1_neigh, b1r)
    (z1,) = _make_sc_agg()(y1, src_r, dst_r)
    s2, y2 = _tc_mid(s1, z1, deg, W2_self, W2_neigh, b2r)
    (z2,) = _make_sc_agg()(y2, src_r, dst_r)
    return _tc_post(s2, z2, deg)


# speedup vs baseline: 3.6054x; 1.1946x over previous
import functools

import jax
import jax.numpy as jnp
from jax import lax
from jax.experimental import pallas as pl
from jax.experimental.pallas import tpu as pltpu
from jax.experimental.pallas import tpu_sc as plsc

N = 10000
E = 320000
D = 128

NC = 2
NS = 16
NW = NC * NS
K = 128
CH = 80
EPP = CH * K
EP = NW * EPP
NP = 10240
RPS = NP // NS
ZR = 128
BR = 1000
GRID = N // BR


def _sc_deg_body(dst_hbm, deg_out, deg_sh, dst_v, rows_v):
    c = lax.axis_index("c")
    s = lax.axis_index("s")
    wid = s * NC + c
    row0 = s * RPS

    zv = jnp.zeros((16,), jnp.float32)

    def fill_rows(val):
        def row(i, _):
            def inner(j, _):
                rows_v[i, pl.ds(j * 16, 16)] = val
                return 0
            return lax.fori_loop(0, D // 16, inner, 0)
        lax.fori_loop(0, K, row, 0)

    fill_rows(zv)
    for t in range(RPS // K):
        pltpu.sync_copy(rows_v, deg_sh.at[pl.ds(row0 + t * K, K)])
    plsc.subcore_barrier()

    fill_rows(jnp.ones((16,), jnp.float32))
    pltpu.sync_copy(dst_hbm.at[wid], dst_v)

    def step(i, _):
        pltpu.sync_copy(rows_v, deg_sh.at[dst_v.at[i]], add=True)
        return 0

    lax.fori_loop(0, CH, step, 0)
    plsc.subcore_barrier()

    pltpu.sync_copy(deg_sh.at[pl.ds(row0, RPS)],
                    deg_out.at[c, pl.ds(row0, RPS)])


def _sc_agg_body(y_hbm, packed_hbm, z_out,
                 z_sh, packed_v, src0, src1, dst0, dst1, rows0, rows1,
                 sem0, sem1):
    c = lax.axis_index("c")
    s = lax.axis_index("s")
    wid = s * NC + c
    row0 = s * RPS

    zv = jnp.zeros((16,), jnp.float32)

    def zero_rows_row(i, _):
        def inner(j, _):
            rows0[i, pl.ds(j * 16, 16)] = zv
            return 0
        return lax.fori_loop(0, D // 16, inner, 0)

    lax.fori_loop(0, K, zero_rows_row, 0)

    for t in range(RPS // K):
        pltpu.sync_copy(rows0, z_sh.at[pl.ds(row0 + t * K, K)])
    plsc.subcore_barrier()

    pltpu.sync_copy(packed_hbm.at[wid], packed_v)

    srcb = (src0, src1)
    dstb = (dst0, dst1)
    rowsb = (rows0, rows1)
    semb = (sem0, sem1)

    def unpack(j, sref, dref):
        def body_u(u, _):
            v = packed_v[j, pl.ds(u * 16, 16)]
            sref[pl.ds(u * 16, 16)] = lax.shift_right_logical(v, 14)
            dref[pl.ds(u * 16, 16)] = lax.bitwise_and(v, 16383)
            return 0
        lax.fori_loop(0, K // 16, body_u, 0)

    for b in range(2):
        unpack(b, srcb[b], dstb[b])
        pltpu.async_copy(y_hbm.at[srcb[b]], rowsb[b], semb[b])

    def outer(t, _):
        for b in range(2):
            i = t * 2 + b
            pltpu.make_async_copy(y_hbm.at[srcb[b]], rowsb[b], semb[b]).wait()
            pltpu.sync_copy(rowsb[b], z_sh.at[dstb[b]], add=True)
            nxt = i + 2

            @pl.when(nxt < CH)
            def _():
                unpack(nxt, srcb[b], dstb[b])
                pltpu.async_copy(y_hbm.at[srcb[b]], rowsb[b], semb[b])
        return 0

    lax.fori_loop(0, CH // 2, outer, 0)
    plsc.subcore_barrier()

    pltpu.sync_copy(z_sh.at[pl.ds(row0, RPS)],
                    z_out.at[c, pl.ds(row0, RPS)])


def _sc_mesh():
    return plsc.VectorSubcoreMesh(core_axis_name="c", subcore_axis_name="s")


@functools.lru_cache(maxsize=None)
def _make_sc_deg():
    return pl.kernel(
        _sc_deg_body,
        mesh=_sc_mesh(),
        out_type=[jax.ShapeDtypeStruct((NC, NP, D), jnp.float32)],
        scratch_types=[
            pltpu.VMEM_SHARED((NP, D), jnp.float32),
            pltpu.VMEM((CH, K), jnp.int32),
            pltpu.VMEM((K, D), jnp.float32),
        ],
    )


@functools.lru_cache(maxsize=None)
def _make_sc_agg():
    return pl.kernel(
        _sc_agg_body,
        mesh=_sc_mesh(),
        out_type=[jax.ShapeDtypeStruct((NC, NP, D), jnp.float32)],
        scratch_types=[
            pltpu.VMEM_SHARED((NP, D), jnp.float32),
            pltpu.VMEM((CH, K), jnp.int32),
            pltpu.VMEM((K,), jnp.int32),
            pltpu.VMEM((K,), jnp.int32),
            pltpu.VMEM((K,), jnp.int32),
            pltpu.VMEM((K,), jnp.int32),
            pltpu.VMEM((K, D), jnp.float32),
            pltpu.VMEM((K, D), jnp.float32),
            pltpu.SemaphoreType.DMA,
            pltpu.SemaphoreType.DMA,
        ],
    )


def _tc_pre_body(x_ref, ws_ref, wn_ref, b_ref, s_out, y_out):
    x = x_ref[...]
    s_out[...] = (jnp.dot(x, ws_ref[...], preferred_element_type=jnp.float32)
                  + b_ref[0:1, :])
    y_out[...] = jnp.dot(x, wn_ref[...], preferred_element_type=jnp.float32)


def _tc_mid_body(s1_ref, z_ref, deg_ref, ws_ref, wn_ref, b_ref, s2_out, y2_out):
    deg = deg_ref[0, :, 0:1] + deg_ref[1, :, 0:1]
    inv = 1.0 / jnp.maximum(deg, 1.0)
    h = jnp.maximum(s1_ref[...] + (z_ref[0] + z_ref[1]) * inv, 0.0)
    s2_out[...] = (jnp.dot(h, ws_ref[...], preferred_element_type=jnp.float32)
                   + b_ref[0:1, :])
    y2_out[...] = jnp.dot(h, wn_ref[...], preferred_element_type=jnp.float32)


def _tc_post_body(s2_ref, z_ref, deg_ref, out_ref):
    deg = deg_ref[0, :, 0:1] + deg_ref[1, :, 0:1]
    inv = 1.0 / jnp.maximum(deg, 1.0)
    out_ref[...] = s2_ref[...] + (z_ref[0] + z_ref[1]) * inv


_row_spec = pl.BlockSpec((BR, D), lambda i: (i, 0))
_w_spec = pl.BlockSpec((D, D), lambda i: (0, 0))
_b_spec = pl.BlockSpec((8, D), lambda i: (0, 0))
_z_spec = pl.BlockSpec((NC, BR, D), lambda i: (0, i, 0))

_tc_pre = pl.pallas_call(
    _tc_pre_body,
    grid=(GRID,),
    in_specs=[_row_spec, _w_spec, _w_spec, _b_spec],
    out_specs=[_row_spec, _row_spec],
    out_shape=[jax.ShapeDtypeStruct((N, D), jnp.float32)] * 2,
)

_tc_mid = pl.pallas_call(
    _tc_mid_body,
    grid=(GRID,),
    in_specs=[_row_spec, _z_spec, _z_spec, _w_spec, _w_spec, _b_spec],
    out_specs=[_row_spec, _row_spec],
    out_shape=[jax.ShapeDtypeStruct((N, D), jnp.float32)] * 2,
)

_tc_post = pl.pallas_call(
    _tc_post_body,
    grid=(GRID,),
    in_specs=[_row_spec, _z_spec, _z_spec],
    out_specs=_row_spec,
    out_shape=jax.ShapeDtypeStruct((N, D), jnp.float32),
)


def kernel(features, edge_index, W1_self, W1_neigh, b1, W2_self, W2_neigh, b2):
    pad = EP - E
    src_p = jnp.concatenate([edge_index[0], jnp.zeros((pad,), jnp.int32)])
    dst_p = jnp.concatenate([edge_index[1], jnp.full((pad,), N, jnp.int32)])
    packed_r = (src_p * 16384 + dst_p).reshape(NW, CH, K)
    dst_r = dst_p.reshape(NW, CH, K)
    b1r = jnp.broadcast_to(b1.reshape(1, D), (8, D))
    b2r = jnp.broadcast_to(b2.reshape(1, D), (8, D))

    (deg,) = _make_sc_deg()(dst_r)
    s1, y1 = _tc_pre(features, W1_self, W1_neigh, b1r)
    (z1,) = _make_sc_agg()(y1, packed_r)
    s2, y2 = _tc_mid(s1, z1, deg, W2_self, W2_neigh, b2r)
    (z2,) = _make_sc_agg()(y2, packed_r)
    return _tc_post(s2, z2, deg)


# speedup vs baseline: 4.0940x; 1.1355x over previous
import functools

import jax
import jax.numpy as jnp
from jax import lax
from jax.experimental import pallas as pl
from jax.experimental.pallas import tpu as pltpu
from jax.experimental.pallas import tpu_sc as plsc

N = 10000
E = 320000
D = 128

NC = 2
NS = 16
NW = NC * NS
K = 128
CH = 80
EPP = CH * K
EP = NW * EPP
CH0 = 128
CH1 = 32
WIN = 32
CPAD = NS * CH0 + NS * CH1

NP = 10240
RPS = NP // NS
ZR = 128
BR = 1000
GRID = N // BR


def _sc_deg_body(dst_hbm, zeros_hbm, ones_hbm, deg_out, deg_sh, dst_v, rows_v):
    c = lax.axis_index("c")
    s = lax.axis_index("s")
    wid = s * NC + c
    row0 = s * RPS

    pltpu.sync_copy(zeros_hbm, deg_sh.at[pl.ds(row0, RPS)])
    plsc.subcore_barrier()

    pltpu.sync_copy(ones_hbm, rows_v)
    pltpu.sync_copy(dst_hbm.at[wid], dst_v)

    def step(i, _):
        pltpu.sync_copy(rows_v, deg_sh.at[dst_v.at[i]], add=True)
        return 0

    lax.fori_loop(0, CH, step, 0)
    plsc.subcore_barrier()

    pltpu.sync_copy(deg_sh.at[pl.ds(row0, RPS)],
                    deg_out.at[c, pl.ds(row0, RPS)])


def _sc_agg_body(y_hbm, src_hbm, dst_hbm, zeros_hbm, z_out,
                 z_sh, src_w, dst_w, rows0, rows1, sem0, sem1):
    c = lax.axis_index("c")
    s = lax.axis_index("s")
    row0 = s * RPS
    nw = jnp.where(c == 0, CH0 // WIN, CH1 // WIN)
    base = jnp.where(c == 0, s * CH0, NS * CH0 + s * CH1)

    pltpu.sync_copy(zeros_hbm, z_sh.at[pl.ds(row0, RPS)])
    plsc.subcore_barrier()

    rowsb = (rows0, rows1)
    semb = (sem0, sem1)

    def window(w, _):
        base_w = base + w * WIN
        pltpu.sync_copy(src_hbm.at[pl.ds(base_w, WIN)], src_w)
        pltpu.sync_copy(dst_hbm.at[pl.ds(base_w, WIN)], dst_w)
        for b in range(2):
            pltpu.async_copy(y_hbm.at[src_w.at[b]], rowsb[b], semb[b])

        def steps(t, _):
            for b in range(2):
                j = t * 2 + b
                pltpu.make_async_copy(y_hbm.at[src_w.at[j]],
                                      rowsb[b], semb[b]).wait()
                pltpu.sync_copy(rowsb[b], z_sh.at[dst_w.at[j]], add=True)
                nxt = j + 2

                @pl.when(nxt < WIN)
                def _():
                    pltpu.async_copy(y_hbm.at[src_w.at[nxt]],
                                     rowsb[b], semb[b])
            return 0

        lax.fori_loop(0, WIN // 2, steps, 0)
        return 0

    lax.fori_loop(0, nw, window, 0)
    plsc.subcore_barrier()

    pltpu.sync_copy(z_sh.at[pl.ds(row0, RPS)],
                    z_out.at[c, pl.ds(row0, RPS)])


def _sc_mesh():
    return plsc.VectorSubcoreMesh(core_axis_name="c", subcore_axis_name="s")


@functools.lru_cache(maxsize=None)
def _make_sc_deg():
    return pl.kernel(
        _sc_deg_body,
        mesh=_sc_mesh(),
        out_type=[jax.ShapeDtypeStruct((NC, NP, D), jnp.float32)],
        scratch_types=[
            pltpu.VMEM_SHARED((NP, D), jnp.float32),
            pltpu.VMEM((CH, K), jnp.int32),
            pltpu.VMEM((K, D), jnp.float32),
        ],
    )


@functools.lru_cache(maxsize=None)
def _make_sc_agg():
    return pl.kernel(
        _sc_agg_body,
        mesh=_sc_mesh(),
        out_type=[jax.ShapeDtypeStruct((NC, NP, D), jnp.float32)],
        scratch_types=[
            pltpu.VMEM_SHARED((NP, D), jnp.float32),
            pltpu.VMEM((WIN, K), jnp.int32),
            pltpu.VMEM((WIN, K), jnp.int32),
            pltpu.VMEM((K, D), jnp.float32),
            pltpu.VMEM((K, D), jnp.float32),
            pltpu.SemaphoreType.DMA,
            pltpu.SemaphoreType.DMA,
        ],
    )


def _tc_pre_body(x_ref, ws_ref, wn_ref, b_ref, s_out, y_out):
    x = x_ref[...]
    s_out[...] = (jnp.dot(x, ws_ref[...], preferred_element_type=jnp.float32)
                  + b_ref[0:1, :])
    y_out[...] = jnp.dot(x, wn_ref[...], preferred_element_type=jnp.float32)


def _tc_mid_body(s1_ref, z_ref, deg_ref, ws_ref, wn_ref, b_ref, s2_out, y2_out):
    deg = deg_ref[0, :, 0:1] + deg_ref[1, :, 0:1]
    inv = 1.0 / jnp.maximum(deg, 1.0)
    h = jnp.maximum(s1_ref[...] + (z_ref[0] + z_ref[1]) * inv, 0.0)
    s2_out[...] = (jnp.dot(h, ws_ref[...], preferred_element_type=jnp.float32)
                   + b_ref[0:1, :])
    y2_out[...] = jnp.dot(h, wn_ref[...], preferred_element_type=jnp.float32)


def _tc_post_body(s2_ref, z_ref, deg_ref, out_ref):
    deg = deg_ref[0, :, 0:1] + deg_ref[1, :, 0:1]
    inv = 1.0 / jnp.maximum(deg, 1.0)
    out_ref[...] = s2_ref[...] + (z_ref[0] + z_ref[1]) * inv


_row_spec = pl.BlockSpec((BR, D), lambda i: (i, 0))
_w_spec = pl.BlockSpec((D, D), lambda i: (0, 0))
_b_spec = pl.BlockSpec((8, D), lambda i: (0, 0))
_z_spec = pl.BlockSpec((NC, BR, D), lambda i: (0, i, 0))

_tc_pre = pl.pallas_call(
    _tc_pre_body,
    grid=(GRID,),
    in_specs=[_row_spec, _w_spec, _w_spec, _b_spec],
    out_specs=[_row_spec, _row_spec],
    out_shape=[jax.ShapeDtypeStruct((N, D), jnp.float32)] * 2,
)

_tc_mid = pl.pallas_call(
    _tc_mid_body,
    grid=(GRID,),
    in_specs=[_row_spec, _z_spec, _z_spec, _w_spec, _w_spec, _b_spec],
    out_specs=[_row_spec, _row_spec],
    out_shape=[jax.ShapeDtypeStruct((N, D), jnp.float32)] * 2,
)

_tc_post = pl.pallas_call(
    _tc_post_body,
    grid=(GRID,),
    in_specs=[_row_spec, _z_spec, _z_spec],
    out_specs=_row_spec,
    out_shape=jax.ShapeDtypeStruct((N, D), jnp.float32),
)


def kernel(features, edge_index, W1_self, W1_neigh, b1, W2_self, W2_neigh, b2):
    pad = CPAD * K - E
    src_p = jnp.concatenate([edge_index[0], jnp.zeros((pad,), jnp.int32)])
    dst_p = jnp.concatenate([edge_index[1], jnp.full((pad,), N, jnp.int32)])
    src_r = src_p.reshape(CPAD, K)
    dst_r2 = dst_p.reshape(CPAD, K)
    dst_r = dst_p[:EP].reshape(NW, CH, K)
    zeros_c = jnp.zeros((RPS, D), jnp.float32)
    ones_c = jnp.ones((K, D), jnp.float32)
    b1r = jnp.broadcast_to(b1.reshape(1, D), (8, D))
    b2r = jnp.broadcast_to(b2.reshape(1, D), (8, D))

    (deg,) = _make_sc_deg()(dst_r, zeros_c, ones_c)
    s1, y1 = _tc_pre(features, W1_self, W1_neigh, b1r)
    (z1,) = _make_sc_agg()(y1, src_r, dst_r2, zeros_c)
    s2, y2 = _tc_mid(s1, z1, deg, W2_self, W2_neigh, b2r)
    (z2,) = _make_sc_agg()(y2, src_r, dst_r2, zeros_c)
    return _tc_post(s2, z2, deg)
